# baseline (speedup 1.0000x reference)
import functools

import jax
import jax.numpy as jnp
from jax import lax
from jax.experimental import pallas as pl
from jax.experimental.pallas import tpu as pltpu
from jax.experimental.pallas import tpu_sc as plsc

DIM = 16
NUM_FEAT = 26
BATCH = 16384
NC, NS, L = 2, 16, 16
NW = NC * NS
B_PER_W = BATCH // NW
G = 64
NQ = B_PER_W // G

ROWS = 999986
RPAD = 1000064
TCH = 128
NCHUNK = RPAD // TCH
CHUNK_LO = NCHUNK // NW
CHUNK_EXTRA = NCHUNK - CHUNK_LO * NW
NBUF = 5
NROUND = (CHUNK_LO + NBUF) // NBUF

_SC_PARAMS = pltpu.CompilerParams(
    use_tc_tiling_on_sc=False, needs_layout_passes=False)
_MESH = dict(core_axis_name="c", subcore_axis_name="s")


def _sc_transpose(tTp):
    @functools.partial(
        pl.kernel,
        mesh=plsc.VectorSubcoreMesh(**_MESH),
        out_type=jax.ShapeDtypeStruct((RPAD, DIM), jnp.float32),
        compiler_params=_SC_PARAMS,
        scratch_types=[
            pltpu.VMEM((NBUF, DIM, TCH), jnp.float32),
            pltpu.VMEM((NBUF, TCH, DIM), jnp.float32),
            pltpu.SemaphoreType.DMA((NBUF,)),
            pltpu.SemaphoreType.DMA((NBUF,)),
        ],
    )
    def k(tTp_hbm, out_hbm, in_v, rows_v, gsem, wsem):
        wid = lax.axis_index("s") * NC + lax.axis_index("c")
        count = CHUNK_LO + jnp.where(wid < CHUNK_EXTRA, 1, 0)
        base = wid * CHUNK_LO + jnp.minimum(wid, CHUNK_EXTRA)
        iota = lax.iota(jnp.int32, L)

        def in_slice(b, ql):
            c0 = (base + ql) * TCH
            return tTp_hbm.at[:, pl.ds(c0, TCH)], in_v.at[b], gsem.at[b]

        def out_slice(b, ql):
            c0 = (base + ql) * TCH
            return rows_v.at[b], out_hbm.at[pl.ds(c0, TCH)], wsem.at[b]

        def round_body(j, carry):
            for b in range(NBUF):
                ql = j * NBUF + b

                @pl.when(ql < count)
                def _():
                    pltpu.async_copy(*in_slice(b, ql))

            for b in range(NBUF):
                ql = j * NBUF + b

                @pl.when(ql < count)
                def _():
                    pltpu.make_async_copy(*in_slice(b, ql)).wait()

                    @pl.when(j > 0)
                    def _():
                        pltpu.make_async_copy(*out_slice(b, ql)).wait()

                    in_b = in_v.at[b]
                    for c in range(TCH):
                        v = plsc.load_gather(
                            in_b, [iota, jnp.full((L,), c, jnp.int32)])
                        rows_v[b, c, :] = v
                    pltpu.async_copy(*out_slice(b, ql))
            return carry

        lax.fori_loop(0, NROUND, round_body, 0)

        for b in range(NBUF):
            ql = (NROUND - 1) * NBUF + b

            @pl.when(ql < count)
            def _():
                pltpu.make_async_copy(*out_slice(b, ql)).wait()

    return k(tTp)


def _sc_embed(xT, off2, table_rm):
    @functools.partial(
        pl.kernel,
        mesh=plsc.VectorSubcoreMesh(**_MESH),
        out_type=jax.ShapeDtypeStruct((NUM_FEAT, DIM, BATCH), jnp.float32),
        compiler_params=_SC_PARAMS,
        scratch_types=[
            pltpu.VMEM((NUM_FEAT, B_PER_W), jnp.int32),
            pltpu.VMEM((NUM_FEAT, L), jnp.int32),
            pltpu.VMEM((2, NUM_FEAT, G, DIM), jnp.float32),
            pltpu.VMEM((2, NUM_FEAT, DIM, G), jnp.float32),
            pltpu.SemaphoreType.DMA,
            pltpu.SemaphoreType.DMA,
            pltpu.SemaphoreType.DMA,
            pltpu.SemaphoreType.DMA,
        ],
    )
    def k(xT_hbm, off2_hbm, table_hbm, out_hbm, idx_v, off_v, rows_v, trows_v,
          gsem0, gsem1, wsem0, wsem1):
        gsems = (gsem0, gsem1)
        wsems = (wsem0, wsem1)
        wid = lax.axis_index("s") * NC + lax.axis_index("c")
        b0 = wid * B_PER_W

        pltpu.sync_copy(xT_hbm.at[:, pl.ds(b0, B_PER_W)], idx_v)
        pltpu.sync_copy(off2_hbm, off_v)

        def add_body(f, carry):
            off = off_v[f, :]
            for c in range(B_PER_W // L):
                sl = pl.ds(c * L, L)
                idx_v[f, sl] = idx_v[f, sl] + off
            return carry

        lax.fori_loop(0, NUM_FEAT, add_body, 0)

        iota = lax.iota(jnp.int32, L)

        def fire_gathers(q):
            b = q % 2
            return [
                pltpu.async_copy(
                    table_hbm.at[idx_v.at[f, pl.ds(q * G, G)]],
                    rows_v.at[b, f],
                    gsems[b],
                )
                for f in range(NUM_FEAT)
            ]

        def transpose_chunk(b):
            def body(f, carry):
                rows_f = rows_v.at[b, f]
                for d in range(DIM):
                    idx1 = jnp.full((L,), d, jnp.int32)
                    for c in range(G // L):
                        v = plsc.load_gather(rows_f, [iota + (c * L), idx1])
                        trows_v[b, f, d, pl.ds(c * L, L)] = v
                return carry
            lax.fori_loop(0, NUM_FEAT, body, 0)

        pending_g = fire_gathers(0)
        pending_w = [None, None]
        for q in range(NQ):
            b = q % 2
            if q + 1 < NQ:
                next_g = fire_gathers(q + 1)
            for h in pending_g:
                h.wait()
            if pending_w[b] is not None:
                pending_w[b].wait()
            transpose_chunk(b)
            if q + 1 < NQ:
                pending_g = next_g
            pending_w[b] = pltpu.async_copy(
                trows_v.at[b],
                out_hbm.at[:, :, pl.ds(b0 + q * G, G)],
                wsems[b],
            )
        pending_w[0].wait()
        pending_w[1].wait()

    return k(xT, off2, table_rm)


def kernel(x_cat, table, offsets):
    xT = x_cat.astype(jnp.int32).T
    off2 = jnp.broadcast_to(offsets.astype(jnp.int32)[:, None], (NUM_FEAT, L))
    tTp = jnp.pad(table.T, ((0, 0), (0, RPAD - ROWS)))
    table_rm = _sc_transpose(tTp)
    out3 = _sc_embed(xT, off2, table_rm)
    return jnp.transpose(out3, (2, 0, 1))


# speedup vs baseline: 1.0098x; 1.0098x over previous
import functools

import jax
import jax.numpy as jnp
from jax import lax
from jax.experimental import pallas as pl
from jax.experimental.pallas import tpu as pltpu
from jax.experimental.pallas import tpu_sc as plsc

DIM = 16
NUM_FEAT = 26
BATCH = 16384
NC, NS, L = 2, 16, 16
NW = NC * NS
B_PER_W = BATCH // NW
G = 64
NQ = B_PER_W // G

ROWS = 999986
TCH = 128
NCHUNK = ROWS // TCH
TAIL0 = NCHUNK * TCH
TAIL = 48
REM0 = TAIL0 + TAIL
REM = ROWS - REM0
CHUNK_LO = NCHUNK // NW
CHUNK_EXTRA = NCHUNK - CHUNK_LO * NW
NBUF = 5
NROUND = (CHUNK_LO + NBUF) // NBUF

_SC_PARAMS = pltpu.CompilerParams(
    use_tc_tiling_on_sc=False, needs_layout_passes=False)
_MESH = dict(core_axis_name="c", subcore_axis_name="s")


def _sc_transpose(tT, tail2):
    @functools.partial(
        pl.kernel,
        mesh=plsc.VectorSubcoreMesh(**_MESH),
        out_type=jax.ShapeDtypeStruct((ROWS, DIM), jnp.float32),
        compiler_params=_SC_PARAMS,
        scratch_types=[
            pltpu.VMEM((NBUF, DIM, TCH), jnp.float32),
            pltpu.VMEM((NBUF, TCH, DIM + 1), jnp.float32),
            pltpu.SemaphoreType.DMA((NBUF,)),
            pltpu.SemaphoreType.DMA((NBUF,)),
        ],
    )
    def k(tT_hbm, tail2_hbm, out_hbm, in_v, rows_v, gsem, wsem):
        wid = lax.axis_index("s") * NC + lax.axis_index("c")
        count = CHUNK_LO + jnp.where(wid < CHUNK_EXTRA, 1, 0)
        base = wid * CHUNK_LO + jnp.minimum(wid, CHUNK_EXTRA)
        iota = lax.iota(jnp.int32, L)

        def c0_of(ql):
            return (base + ql) * TCH

        def in_slice(b, ql):
            return tT_hbm.at[:, pl.ds(c0_of(ql), TCH)], in_v.at[b], gsem.at[b]

        def out_slice(b, ql):
            return (rows_v.at[b, :, pl.ds(0, DIM)],
                    out_hbm.at[pl.ds(c0_of(ql), TCH)], wsem.at[b])

        def round_body(j, carry):
            for b in range(NBUF):
                ql = j * NBUF + b

                @pl.when(ql < count)
                def _():
                    pltpu.async_copy(*in_slice(b, ql))

            for b in range(NBUF):
                ql = j * NBUF + b

                @pl.when(ql < count)
                def _():
                    pltpu.make_async_copy(*in_slice(b, ql)).wait()

                    @pl.when(j > 0)
                    def _():
                        pltpu.make_async_copy(*out_slice(b, ql)).wait()

                    in_b = in_v.at[b]
                    rows_b = rows_v.at[b]
                    for d in range(DIM):
                        idx1 = jnp.full((L,), d, jnp.int32)
                        for c in range(TCH // L):
                            v = in_b[d, pl.ds(c * L, L)]
                            plsc.store_scatter(rows_b, [iota + (c * L), idx1], v)
                    pltpu.async_copy(*out_slice(b, ql))
            return carry

        lax.fori_loop(0, NROUND, round_body, 0)

        for b in range(NBUF):
            ql = (NROUND - 1) * NBUF + b

            @pl.when(ql < count)
            def _():
                pltpu.make_async_copy(*out_slice(b, ql)).wait()

        @pl.when(wid == NW - 1)
        def _():
            pltpu.sync_copy(tT_hbm.at[:, pl.ds(TAIL0, TAIL)],
                            in_v.at[0, :, pl.ds(0, TAIL)])
            rows_b = rows_v.at[0]
            for d in range(DIM):
                idx1 = jnp.full((L,), d, jnp.int32)
                for c in range(TAIL // L):
                    v = in_v[0, d, pl.ds(c * L, L)]
                    plsc.store_scatter(rows_b, [iota + (c * L), idx1], v)
            pltpu.sync_copy(rows_v.at[0, pl.ds(0, TAIL), pl.ds(0, DIM)],
                            out_hbm.at[pl.ds(TAIL0, TAIL)])
            pltpu.sync_copy(tail2_hbm, out_hbm.at[pl.ds(REM0, REM)])

    return k(tT, tail2)


def _sc_embed(xT, off2, table_rm):
    @functools.partial(
        pl.kernel,
        mesh=plsc.VectorSubcoreMesh(**_MESH),
        out_type=jax.ShapeDtypeStruct((NUM_FEAT, DIM, BATCH), jnp.float32),
        compiler_params=_SC_PARAMS,
        scratch_types=[
            pltpu.VMEM((NUM_FEAT, B_PER_W), jnp.int32),
            pltpu.VMEM((NUM_FEAT, L), jnp.int32),
            pltpu.VMEM((2, NUM_FEAT, G, DIM), jnp.float32),
            pltpu.VMEM((2, NUM_FEAT, DIM, G + 1), jnp.float32),
            pltpu.SemaphoreType.DMA,
            pltpu.SemaphoreType.DMA,
            pltpu.SemaphoreType.DMA,
            pltpu.SemaphoreType.DMA,
        ],
    )
    def k(xT_hbm, off2_hbm, table_hbm, out_hbm, idx_v, off_v, rows_v, trows_v,
          gsem0, gsem1, wsem0, wsem1):
        gsems = (gsem0, gsem1)
        wsems = (wsem0, wsem1)
        wid = lax.axis_index("s") * NC + lax.axis_index("c")
        b0 = wid * B_PER_W

        pltpu.sync_copy(xT_hbm.at[:, pl.ds(b0, B_PER_W)], idx_v)
        pltpu.sync_copy(off2_hbm, off_v)

        def add_body(f, carry):
            off = off_v[f, :]
            for c in range(B_PER_W // L):
                sl = pl.ds(c * L, L)
                idx_v[f, sl] = idx_v[f, sl] + off
            return carry

        lax.fori_loop(0, NUM_FEAT, add_body, 0)

        iota = lax.iota(jnp.int32, L)

        def fire_gathers(q):
            b = q % 2
            return [
                pltpu.async_copy(
                    table_hbm.at[idx_v.at[f, pl.ds(q * G, G)]],
                    rows_v.at[b, f],
                    gsems[b],
                )
                for f in range(NUM_FEAT)
            ]

        def transpose_chunk(b):
            def body(f, carry):
                rows_f = rows_v.at[b, f]
                trows_f = trows_v.at[b, f]
                for r in range(G):
                    v = rows_f[r, :]
                    plsc.store_scatter(
                        trows_f, [iota, jnp.full((L,), r, jnp.int32)], v)
                return carry
            lax.fori_loop(0, NUM_FEAT, body, 0)

        pending_g = fire_gathers(0)
        pending_w = [None, None]
        for q in range(NQ):
            b = q % 2
            if q + 1 < NQ:
                next_g = fire_gathers(q + 1)
            for h in pending_g:
                h.wait()
            if pending_w[b] is not None:
                pending_w[b].wait()
            transpose_chunk(b)
            if q + 1 < NQ:
                pending_g = next_g
            pending_w[b] = pltpu.async_copy(
                trows_v.at[b, :, :, pl.ds(0, G)],
                out_hbm.at[:, :, pl.ds(b0 + q * G, G)],
                wsems[b],
            )
        pending_w[0].wait()
        pending_w[1].wait()

    return k(xT, off2, table_rm)


def kernel(x_cat, table, offsets):
    xT = x_cat.astype(jnp.int32).T
    off2 = jnp.broadcast_to(offsets.astype(jnp.int32)[:, None], (NUM_FEAT, L))
    table_rm = _sc_transpose(table.T, table[REM0:, :])
    out3 = _sc_embed(xT, off2, table_rm)
    return jnp.transpose(out3, (2, 0, 1))


# speedup vs baseline: 3.1936x; 3.1627x over previous
import functools

import jax
import jax.numpy as jnp
from jax import lax
from jax.experimental import pallas as pl
from jax.experimental.pallas import tpu as pltpu
from jax.experimental.pallas import tpu_sc as plsc

DIM = 16
NUM_FEAT = 26
BATCH = 16384
NC, NS, L = 2, 16, 16
NW = NC * NS
B_PER_W = BATCH // NW
G = 64
NQ = B_PER_W // G


def _sc_embed(xT, off2, table):
    mesh = plsc.VectorSubcoreMesh(core_axis_name="c", subcore_axis_name="s")

    @functools.partial(
        pl.kernel,
        mesh=mesh,
        out_type=jax.ShapeDtypeStruct((NUM_FEAT, DIM, BATCH), jnp.float32),
        compiler_params=pltpu.CompilerParams(
            use_tc_tiling_on_sc=False, needs_layout_passes=False),
        scratch_types=[
            pltpu.VMEM((NUM_FEAT, B_PER_W), jnp.int32),
            pltpu.VMEM((NUM_FEAT, L), jnp.int32),
            pltpu.VMEM((2, NUM_FEAT, G, DIM), jnp.float32),
            pltpu.VMEM((2, NUM_FEAT, DIM, G + 1), jnp.float32),
            pltpu.SemaphoreType.DMA,
            pltpu.SemaphoreType.DMA,
            pltpu.SemaphoreType.DMA,
            pltpu.SemaphoreType.DMA,
        ],
    )
    def k(xT_hbm, off2_hbm, table_hbm, out_hbm, idx_v, off_v, rows_v, trows_v,
          gsem0, gsem1, wsem0, wsem1):
        gsems = (gsem0, gsem1)
        wsems = (wsem0, wsem1)
        wid = lax.axis_index("s") * NC + lax.axis_index("c")
        b0 = wid * B_PER_W

        pltpu.sync_copy(xT_hbm.at[:, pl.ds(b0, B_PER_W)], idx_v)
        pltpu.sync_copy(off2_hbm, off_v)

        def add_body(f, carry):
            off = off_v[f, :]
            for c in range(B_PER_W // L):
                sl = pl.ds(c * L, L)
                idx_v[f, sl] = idx_v[f, sl] + off
            return carry

        lax.fori_loop(0, NUM_FEAT, add_body, 0)

        iota = lax.iota(jnp.int32, L)

        def fire_gathers(q):
            b = q % 2
            return [
                pltpu.async_copy(
                    table_hbm.at[idx_v.at[f, pl.ds(q * G, G)]],
                    rows_v.at[b, f],
                    gsems[b],
                )
                for f in range(NUM_FEAT)
            ]

        def transpose_chunk(b):
            def body(f, carry):
                rows_f = rows_v.at[b, f]
                trows_f = trows_v.at[b, f]
                for r in range(G):
                    v = rows_f[r, :]
                    plsc.store_scatter(
                        trows_f, [iota, jnp.full((L,), r, jnp.int32)], v)
                return carry
            lax.fori_loop(0, NUM_FEAT, body, 0)

        pending_g = fire_gathers(0)
        pending_w = [None, None]
        for q in range(NQ):
            b = q % 2
            if q + 1 < NQ:
                next_g = fire_gathers(q + 1)
            for h in pending_g:
                h.wait()
            if pending_w[b] is not None:
                pending_w[b].wait()
            transpose_chunk(b)
            if q + 1 < NQ:
                pending_g = next_g
            pending_w[b] = pltpu.async_copy(
                trows_v.at[b, :, :, pl.ds(0, G)],
                out_hbm.at[:, :, pl.ds(b0 + q * G, G)],
                wsems[b],
            )
        pending_w[0].wait()
        pending_w[1].wait()

    return k(xT, off2, table)


def kernel(x_cat, table, offsets):
    xT = x_cat.astype(jnp.int32).T
    off2 = jnp.broadcast_to(offsets.astype(jnp.int32)[:, None], (NUM_FEAT, L))
    out3 = _sc_embed(xT, off2, table)
    return jnp.transpose(out3, (2, 0, 1))


# speedup vs baseline: 3.1969x; 1.0010x over previous
import functools

import jax
import jax.numpy as jnp
from jax import lax
from jax.experimental import pallas as pl
from jax.experimental.pallas import tpu as pltpu
from jax.experimental.pallas import tpu_sc as plsc

DIM = 16
NUM_FEAT = 26
BATCH = 16384
NC, NS, L = 2, 16, 16
NW = NC * NS
B_PER_W = BATCH // NW
G = 64
NQ = B_PER_W // G


def _sc_embed(xT, off2, table):
    mesh = plsc.VectorSubcoreMesh(core_axis_name="c", subcore_axis_name="s")

    @functools.partial(
        pl.kernel,
        mesh=mesh,
        out_type=jax.ShapeDtypeStruct((NUM_FEAT, DIM, BATCH), jnp.float32),
        compiler_params=pltpu.CompilerParams(
            use_tc_tiling_on_sc=False, needs_layout_passes=False),
        scratch_types=[
            pltpu.VMEM((NUM_FEAT, B_PER_W), jnp.int32),
            pltpu.VMEM((NUM_FEAT, L), jnp.int32),
            pltpu.VMEM((2, NUM_FEAT, G, DIM), jnp.float32),
            pltpu.VMEM((2, NUM_FEAT, DIM, G + 1), jnp.float32),
            pltpu.SemaphoreType.DMA,
            pltpu.SemaphoreType.DMA,
            pltpu.SemaphoreType.DMA,
            pltpu.SemaphoreType.DMA,
        ],
    )
    def k(xT_hbm, off2_hbm, table_hbm, out_hbm, idx_v, off_v, rows_v, trows_v,
          gsem0, gsem1, wsem0, wsem1):
        gsems = (gsem0, gsem1)
        wsems = (wsem0, wsem1)
        wid = lax.axis_index("s") * NC + lax.axis_index("c")
        b0 = wid * B_PER_W

        pltpu.sync_copy(xT_hbm.at[:, pl.ds(b0, B_PER_W)], idx_v)
        pltpu.sync_copy(off2_hbm, off_v)

        def add_chunk(q):
            def add_body(f, carry):
                off = off_v[f, :]
                for c in range(G // L):
                    sl = pl.ds(q * G + c * L, L)
                    idx_v[f, sl] = idx_v[f, sl] + off
                return carry
            lax.fori_loop(0, NUM_FEAT, add_body, 0)

        iota = lax.iota(jnp.int32, L)

        def fire_gathers(q):
            b = q % 2
            return [
                pltpu.async_copy(
                    table_hbm.at[idx_v.at[f, pl.ds(q * G, G)]],
                    rows_v.at[b, f],
                    gsems[b],
                )
                for f in range(NUM_FEAT)
            ]

        def transpose_chunk(b):
            def body(f, carry):
                rows_f = rows_v.at[b, f]
                trows_f = trows_v.at[b, f]
                for r in range(G):
                    v = rows_f[r, :]
                    plsc.store_scatter(
                        trows_f, [iota, jnp.full((L,), r, jnp.int32)], v)
                return carry
            lax.fori_loop(0, NUM_FEAT, body, 0)

        add_chunk(0)
        pending_g = fire_gathers(0)
        pending_w = [None, None]
        for q in range(NQ):
            b = q % 2
            if q + 1 < NQ:
                add_chunk(q + 1)
                next_g = fire_gathers(q + 1)
            for h in pending_g:
                h.wait()
            if pending_w[b] is not None:
                pending_w[b].wait()
            transpose_chunk(b)
            if q + 1 < NQ:
                pending_g = next_g
            pending_w[b] = pltpu.async_copy(
                trows_v.at[b, :, :, pl.ds(0, G)],
                out_hbm.at[:, :, pl.ds(b0 + q * G, G)],
                wsems[b],
            )
        pending_w[0].wait()
        pending_w[1].wait()

    return k(xT, off2, table)


def kernel(x_cat, table, offsets):
    xT = x_cat.astype(jnp.int32).T
    off2 = jnp.broadcast_to(offsets.astype(jnp.int32)[:, None], (NUM_FEAT, L))
    out3 = _sc_embed(xT, off2, table)
    return jnp.transpose(out3, (2, 0, 1))
